# baseline (speedup 1.0000x reference)
import functools

import jax
import jax.numpy as jnp
from jax import lax
from jax.experimental import pallas as pl
from jax.experimental.pallas import tpu as pltpu
from jax.experimental.pallas import tpu_sc as plsc

B = 16384
F = 26
K = 16
FEAT = 100000
NC = 2
NS = 16
NW = NC * NS
BPW = B // NW
G = 64
NCH = BPW // G
NIDX = G * F
IR = NIDX // 128
NROWS_IDX = (BPW * F) // 128
NBUF = 2

_mesh = plsc.VectorSubcoreMesh(core_axis_name="c", subcore_axis_name="s")


@functools.partial(
    pl.kernel,
    out_type=jax.ShapeDtypeStruct((B,), jnp.float32),
    mesh=_mesh,
    compiler_params=pltpu.CompilerParams(
        needs_layout_passes=False, use_tc_tiling_on_sc=False),
    scratch_types=[
        pltpu.VMEM((BPW * F,), jnp.int32),
        pltpu.VMEM((NROWS_IDX, 128), jnp.int32),
        pltpu.VMEM((NBUF, NIDX, K), jnp.float32),
        pltpu.VMEM((NBUF * NIDX,), jnp.float32),
        pltpu.VMEM((256,), jnp.float32),
        pltpu.VMEM((BPW,), jnp.float32),
        pltpu.VMEM((16,), jnp.float32),
        pltpu.SemaphoreType.DMA,
        pltpu.SemaphoreType.DMA,
    ],
)
def _fm_sc(inp_hbm, w0_hbm, w_hbm, v_hbm, out_hbm,
           inp_v, idx_v, vrows, wvals, tbuf, outv, w0v, vsem, wsem):
    cid = lax.axis_index("c")
    sid = lax.axis_index("s")
    wid = sid * NC + cid
    base = wid * BPW
    iota = lax.iota(jnp.int32, 16)
    iotaF = iota * F
    iota16 = iota * 16

    pltpu.sync_copy(w0_hbm, w0v.at[pl.ds(0, 1)])
    pltpu.sync_copy(inp_hbm.at[pl.ds(base * F, BPW * F)], inp_v)

    def build(j, carry):
        p = j * 16 + iota
        vals = inp_v[pl.ds(j * 16, 16)]
        gidx = vals + lax.rem(p, F) * FEAT
        idx_v[j // 8, pl.ds(lax.rem(j, 8) * 16, 16)] = gidx
        return carry

    lax.fori_loop(0, (BPW * F) // 16, build, 0)

    w0s = w0v[pl.ds(0, 16)][0]
    half = jnp.float32(0.5)

    def start_chunk(c, buf):
        for j in range(IR):
            ir = c * IR + j
            pltpu.make_async_copy(
                v_hbm.at[idx_v.at[ir]],
                vrows.at[buf, pl.ds(j * 128, 128)],
                vsem,
            ).start()
            pltpu.make_async_copy(
                w_hbm.at[idx_v.at[ir]],
                wvals.at[pl.ds(buf * NIDX + j * 128, 128)],
                wsem,
            ).start()

    def wait_chunk(c, buf):
        for j in range(IR):
            ir = c * IR + j
            pltpu.make_async_copy(
                v_hbm.at[idx_v.at[ir]],
                vrows.at[buf, pl.ds(j * 128, 128)],
                vsem,
            ).wait()
            pltpu.make_async_copy(
                w_hbm.at[idx_v.at[ir]],
                wvals.at[pl.ds(buf * NIDX + j * 128, 128)],
                wsem,
            ).wait()

    def compute_chunk(c, buf):
        def group(g, carry):
            def row(bi, rcarry):
                rb = (g * 16 + bi) * F
                v0 = vrows[buf, rb, :]
                s = v0
                q = v0 * v0
                for f in range(1, F):
                    v = vrows[buf, rb + f, :]
                    s = s + v
                    q = q + v * v
                tbuf[pl.ds(bi * 16, 16)] = s * s - q
                return rcarry

            lax.fori_loop(0, 16, row, 0)

            acc = plsc.load_gather(tbuf, [iota16])
            for k in range(1, 16):
                acc = acc + plsc.load_gather(tbuf, [iota16 + k])

            wb = buf * NIDX + g * 16 * F
            fw = plsc.load_gather(wvals, [iotaF + wb])
            for f in range(1, F):
                fw = fw + plsc.load_gather(wvals, [iotaF + (wb + f)])

            outv[pl.ds(c * G + g * 16, 16)] = half * acc + fw + w0s
            return carry

        lax.fori_loop(0, G // 16, group, 0)

    def chunk(c, carry):
        buf = lax.rem(c, NBUF)
        start_chunk(c, buf)
        wait_chunk(c, buf)
        compute_chunk(c, buf)
        return carry

    lax.fori_loop(0, NCH, chunk, 0)

    pltpu.sync_copy(outv, out_hbm.at[pl.ds(base, BPW)])


def kernel(inputs, w0, w, V):
    inp_flat = inputs.reshape(-1).astype(jnp.int32)
    w_flat = w.reshape(-1)
    out = _fm_sc(inp_flat, w0, w_flat, V)
    return out.reshape(B, 1)


# speedup vs baseline: 1.6222x; 1.6222x over previous
import functools

import jax
import jax.numpy as jnp
from jax import lax
from jax.experimental import pallas as pl
from jax.experimental.pallas import tpu as pltpu
from jax.experimental.pallas import tpu_sc as plsc

B = 16384
F = 26
K = 16
FEAT = 100000
FLEN = F * FEAT
HW = FEAT // 2
PW = 51200
SCW = PW // 16
NGEN = 2 * F
GB_MAX = 2600064 - PW
WB_MAX = FLEN - PW
NC = 2
NS = 16
BH = B // 2
NVEC = BH // 16
BSL = B // NS

_mesh = plsc.VectorSubcoreMesh(core_axis_name="c", subcore_axis_name="s")


@functools.partial(
    pl.kernel,
    out_type=jax.ShapeDtypeStruct((NC, B), jnp.float32),
    mesh=_mesh,
    compiler_params=pltpu.CompilerParams(needs_layout_passes=False),
    scratch_types=[
        pltpu.VMEM_SHARED((8, PW), jnp.float32),
        pltpu.VMEM_SHARED((8, PW), jnp.float32),
        pltpu.VMEM((PW,), jnp.float32),
        pltpu.VMEM((BH,), jnp.int32),
        pltpu.VMEM((BH,), jnp.float32),
        pltpu.VMEM((BH,), jnp.float32),
        pltpu.VMEM((BSL,), jnp.float32),
        pltpu.VMEM((16,), jnp.float32),
        pltpu.HBM((NC * NS, BH), jnp.float32),
        pltpu.HBM((NC * NS, BH), jnp.float32),
        pltpu.SemaphoreType.DMA,
    ],
)
def _fm_sc(inp_hbm, w0_hbm, w_hbm, v_t_hbm, out_hbm,
           pool0, pool1, win, inprow, s_acc, a_acc, outv, w0v,
           s_scr, a_scr, stsem):
    pools = (pool0, pool1)
    c = lax.axis_index("c")
    s = lax.axis_index("s")
    wid = c * NS + s
    p = lax.rem(s, 8)
    bh = s // 8
    bbase = bh * BH
    c8 = pl.multiple_of(c * 8, 8)
    half = jnp.float32(0.5)

    pltpu.sync_copy(w0_hbm, w0v.at[pl.ds(0, 1)])

    def zero(j, carry):
        z = jnp.zeros((16,), jnp.float32)
        s_acc[pl.ds(j * 16, 16)] = z
        a_acc[pl.ds(j * 16, 16)] = z
        return carry

    lax.fori_loop(0, NVEC, zero, 0)

    def pool_base(g):
        f = g // 2
        hg = lax.rem(g, 2)
        start = f * FEAT + hg * HW
        return jnp.minimum((start // 128) * 128, GB_MAX)

    def stripe_copy(g, buf):
        gb = pl.multiple_of(pool_base(g), 128)
        return pltpu.make_async_copy(
            v_t_hbm.at[pl.ds(c8, 8), pl.ds(gb + s * SCW, SCW)],
            pools[buf].at[:, pl.ds(s * SCW, SCW)],
            stsem,
        )

    def gen_task(g, cur):
        f = g // 2
        hg = lax.rem(g, 2)
        lo = hg * HW

        @pl.when(g + 1 < NGEN)
        def _prefetch():
            stripe_copy(g + 1, 1 - cur).start()

        pltpu.sync_copy(inp_hbm.at[pl.ds(f * B + bbase, BH)], inprow)
        pltpu.sync_copy(pools[cur].at[p], win)

        off = f * FEAT - pool_base(g)

        def body(j, carry):
            idx = inprow[pl.ds(j * 16, 16)]
            mrel = idx - lo
            m = (mrel >= 0) & (mrel < HW)
            v = plsc.load_gather(win, [jnp.where(m, idx + off, 0)])
            v = jnp.where(m, v, jnp.float32(0.0))
            o = pl.ds(j * 16, 16)
            s_acc[o] = s_acc[o] + v
            a_acc[o] = a_acc[o] - half * (v * v)
            return carry

        lax.fori_loop(0, NVEC, body, 0)

        @pl.when((c == hg) & (p == lax.rem(f, 8)))
        def _w_task():
            wb = jnp.minimum(f * FEAT + lo, WB_MAX)
            woff = f * FEAT - wb
            pltpu.sync_copy(w_hbm.at[pl.ds(wb, PW)], win)

            def wbody(j, carry):
                idx = inprow[pl.ds(j * 16, 16)]
                mrel = idx - lo
                m = (mrel >= 0) & (mrel < HW)
                v = plsc.load_gather(win, [jnp.where(m, idx + woff, 0)])
                v = jnp.where(m, v, jnp.float32(0.0))
                o = pl.ds(j * 16, 16)
                a_acc[o] = a_acc[o] + v
                return carry

            lax.fori_loop(0, NVEC, wbody, 0)

        @pl.when(g + 1 < NGEN)
        def _drain():
            stripe_copy(g + 1, 1 - cur).wait()

        plsc.subcore_barrier()

    stripe_copy(0, 0).start()
    stripe_copy(0, 0).wait()
    plsc.subcore_barrier()

    def pair(i, carry):
        gen_task(2 * i, 0)
        gen_task(2 * i + 1, 1)
        return carry

    lax.fori_loop(0, NGEN // 2, pair, 0)

    pltpu.sync_copy(s_acc, s_scr.at[wid])
    pltpu.sync_copy(a_acc, a_scr.at[wid])
    plsc.subcore_barrier()

    bs = s * BSL
    bhm = s // 8
    o8 = lax.rem(s, 8) * BSL
    for q in range(8):
        pltpu.sync_copy(a_scr.at[c * NS + bhm * 8 + q, pl.ds(o8, BSL)],
                        win.at[pl.ds(q * BSL, BSL)])
    for q in range(8):
        pltpu.sync_copy(s_scr.at[c * NS + bhm * 8 + q, pl.ds(o8, BSL)],
                        win.at[pl.ds((8 + q) * BSL, BSL)])

    w0s = w0v[pl.ds(0, 16)][0]
    w0_eff = jnp.where(c == 0, w0s, jnp.float32(0.0))

    def fin(j, carry):
        acc = jnp.full((16,), w0_eff, jnp.float32)
        for q in range(8):
            acc = acc + win[pl.ds(q * BSL + j * 16, 16)]
        for q in range(8):
            sq = win[pl.ds((8 + q) * BSL + j * 16, 16)]
            acc = acc + half * (sq * sq)
        outv[pl.ds(j * 16, 16)] = acc
        return carry

    lax.fori_loop(0, BSL // 16, fin, 0)

    pltpu.sync_copy(outv, out_hbm.at[c, pl.ds(bs, BSL)])


def kernel(inputs, w0, w, V):
    out2 = _fm_sc(inputs.T.reshape(-1), w0, w.reshape(-1), V.T)
    return (out2[0] + out2[1]).reshape(B, 1)


# speedup vs baseline: 2.9012x; 1.7884x over previous
import functools

import jax
import jax.numpy as jnp
from jax import lax
from jax.experimental import pallas as pl
from jax.experimental.pallas import tpu as pltpu
from jax.experimental.pallas import tpu_sc as plsc

B = 16384
F = 26
K = 16
FEAT = 100000
FLEN = F * FEAT
HW = FEAT // 2
PW = 51200
SCW = PW // 16
NGEN = 2 * F
GB_MAX = 2600064 - PW
WB_MAX = FLEN - PW
NC = 2
NS = 16
BH = B // 2
NVEC = BH // 16
BSL = B // NS

_mesh = plsc.VectorSubcoreMesh(core_axis_name="c", subcore_axis_name="s")


@functools.partial(
    pl.kernel,
    out_type=jax.ShapeDtypeStruct((NC, B), jnp.float32),
    mesh=_mesh,
    compiler_params=pltpu.CompilerParams(needs_layout_passes=False),
    scratch_types=[
        pltpu.VMEM_SHARED((8, PW), jnp.float32),
        pltpu.VMEM_SHARED((8, PW), jnp.float32),
        pltpu.VMEM((PW,), jnp.float32),
        pltpu.VMEM((BH,), jnp.int32),
        pltpu.VMEM((BH,), jnp.float32),
        pltpu.VMEM((BH,), jnp.float32),
        pltpu.VMEM((BSL,), jnp.float32),
        pltpu.VMEM((16,), jnp.float32),
        pltpu.HBM((NC * NS, BH), jnp.float32),
        pltpu.HBM((NC * NS, BH), jnp.float32),
        pltpu.SemaphoreType.DMA,
    ],
)
def _fm_sc(inp_hbm, w0_hbm, w_hbm, v_t_hbm, out_hbm,
           pool0, pool1, win, inprow, s_acc, a_acc, outv, w0v,
           s_scr, a_scr, stsem):
    pools = (pool0, pool1)
    c = lax.axis_index("c")
    s = lax.axis_index("s")
    wid = c * NS + s
    p = lax.rem(s, 8)
    bh = s // 8
    bbase = bh * BH
    c8 = pl.multiple_of(c * 8, 8)
    half = jnp.float32(0.5)

    pltpu.sync_copy(w0_hbm, w0v.at[pl.ds(0, 1)])

    @plsc.parallel_loop(0, BH, step=16, unroll=4)
    def _zero(i):
        z = jnp.zeros((16,), jnp.float32)
        s_acc[pl.ds(i, 16)] = z
        a_acc[pl.ds(i, 16)] = z

    def pool_base(g):
        f = g // 2
        hg = lax.rem(g, 2)
        start = f * FEAT + hg * HW
        return jnp.minimum((start // 128) * 128, GB_MAX)

    def stripe_copy(g, buf):
        gb = pl.multiple_of(pool_base(g), 128)
        return pltpu.make_async_copy(
            v_t_hbm.at[pl.ds(c8, 8), pl.ds(gb + s * SCW, SCW)],
            pools[buf].at[:, pl.ds(s * SCW, SCW)],
            stsem,
        )

    def gen_task(g, cur, stage_inp):
        f = g // 2
        hg = lax.rem(g, 2)
        lo = hg * HW

        @pl.when(g + 1 < NGEN)
        def _prefetch():
            stripe_copy(g + 1, 1 - cur).start()

        if stage_inp:
            pltpu.sync_copy(inp_hbm.at[pl.ds(f * B + bbase, BH)], inprow)
        pltpu.sync_copy(pools[cur].at[p], win)

        off = f * FEAT - pool_base(g)

        @plsc.parallel_loop(0, BH, step=16, unroll=4)
        def _body(i):
            idx = inprow[pl.ds(i, 16)]
            mrel = idx - lo
            m = (mrel >= 0) & (mrel < HW)
            v = plsc.load_gather(win, [jnp.where(m, idx + off, 0)])
            v = jnp.where(m, v, jnp.float32(0.0))
            o = pl.ds(i, 16)
            s_acc[o] = s_acc[o] + v
            a_acc[o] = a_acc[o] - half * (v * v)

        @pl.when(g + 1 < NGEN)
        def _drain():
            stripe_copy(g + 1, 1 - cur).wait()

        plsc.subcore_barrier()

    stripe_copy(0, 0).start()
    stripe_copy(0, 0).wait()
    plsc.subcore_barrier()

    def pair(i, carry):
        gen_task(2 * i, 0, stage_inp=True)
        gen_task(2 * i + 1, 1, stage_inp=False)
        return carry

    lax.fori_loop(0, NGEN // 2, pair, 0)

    for r in range(4):
        tid = c * 8 + p + 16 * r

        @pl.when(tid < NGEN)
        def _w_task():
            f = tid // 2
            hg = lax.rem(tid, 2)
            lo = hg * HW
            wb = jnp.minimum(f * FEAT + lo, WB_MAX)
            woff = f * FEAT - wb
            pltpu.sync_copy(inp_hbm.at[pl.ds(f * B + bbase, BH)], inprow)
            pltpu.sync_copy(w_hbm.at[pl.ds(wb, PW)], win)

            @plsc.parallel_loop(0, BH, step=16, unroll=4)
            def _wbody(i):
                idx = inprow[pl.ds(i, 16)]
                mrel = idx - lo
                m = (mrel >= 0) & (mrel < HW)
                v = plsc.load_gather(win, [jnp.where(m, idx + woff, 0)])
                v = jnp.where(m, v, jnp.float32(0.0))
                o = pl.ds(i, 16)
                a_acc[o] = a_acc[o] + v

    pltpu.sync_copy(s_acc, s_scr.at[wid])
    pltpu.sync_copy(a_acc, a_scr.at[wid])
    plsc.subcore_barrier()

    bs = s * BSL
    bhm = s // 8
    o8 = lax.rem(s, 8) * BSL
    for q in range(8):
        pltpu.sync_copy(a_scr.at[c * NS + bhm * 8 + q, pl.ds(o8, BSL)],
                        win.at[pl.ds(q * BSL, BSL)])
    for q in range(8):
        pltpu.sync_copy(s_scr.at[c * NS + bhm * 8 + q, pl.ds(o8, BSL)],
                        win.at[pl.ds((8 + q) * BSL, BSL)])

    w0s = w0v[pl.ds(0, 16)][0]
    w0_eff = jnp.where(c == 0, w0s, jnp.float32(0.0))

    def fin(j, carry):
        acc = jnp.full((16,), w0_eff, jnp.float32)
        for q in range(8):
            acc = acc + win[pl.ds(q * BSL + j * 16, 16)]
        for q in range(8):
            sq = win[pl.ds((8 + q) * BSL + j * 16, 16)]
            acc = acc + half * (sq * sq)
        outv[pl.ds(j * 16, 16)] = acc
        return carry

    lax.fori_loop(0, BSL // 16, fin, 0)

    pltpu.sync_copy(outv, out_hbm.at[c, pl.ds(bs, BSL)])


def kernel(inputs, w0, w, V):
    out2 = _fm_sc(inputs.T.reshape(-1), w0, w.reshape(-1), V.T)
    return (out2[0] + out2[1]).reshape(B, 1)
